# baseline (speedup 1.0000x reference)
import functools

import jax
import jax.numpy as jnp
from jax import lax
from jax.experimental import pallas as pl
from jax.experimental.pallas import tpu as pltpu
from jax.experimental.pallas import tpu_sc as plsc

B, L = 4096, 200
EMB = 64
ID_DIM = EMB - 8
VOCAB = 201

NC, NS = 2, 16
NW = NC * NS
ROWS_W = B // NW
IW = 128
IW2 = L - IW
LANES = 16
GRPF = L // LANES
TAIL = L - GRPF * LANES
CBUF = 208

assert B % NW == 0 and ROWS_W % 2 == 0 and IW2 % 8 == 0


def _prep_body(emb_ref, wc_ref, bc_ref, wf_ref, bf_ref, tbl_ref, v_ref):
    wf = wf_ref[...]
    wf1 = wf[:, :ID_DIM]
    wf2 = wf[:, ID_DIM:]
    const = lax.dot_general(bc_ref[...], wf2, (((1,), (1,)), ((), ())),
                            preferred_element_type=jnp.float32) + bf_ref[...]
    tbl = lax.dot_general(emb_ref[...], wf1, (((1,), (1,)), ((), ())),
                          preferred_element_type=jnp.float32)
    tbl_ref[...] = tbl + const
    v_ref[...] = lax.dot_general(wc_ref[...], wf2, (((0,), (1,)), ((), ())),
                                 preferred_element_type=jnp.float32)


_prep = pl.pallas_call(
    _prep_body,
    out_shape=(
        jax.ShapeDtypeStruct((VOCAB, EMB), jnp.float32),
        jax.ShapeDtypeStruct((1, EMB), jnp.float32),
    ),
)


_sc_mesh = plsc.VectorSubcoreMesh(core_axis_name="c", subcore_axis_name="s")


@functools.partial(
    pl.kernel,
    out_type=jax.ShapeDtypeStruct((B, L, EMB), jnp.float32),
    mesh=_sc_mesh,
    scratch_types=[
        pltpu.VMEM((2, 2, IW), jnp.int32),
        pltpu.VMEM((2, CBUF), jnp.float32),
        pltpu.VMEM((2, L, EMB), jnp.float32),
        pltpu.VMEM_SHARED((VOCAB, EMB), jnp.float32),
        pltpu.VMEM((EMB,), jnp.float32),
        pltpu.SemaphoreType.DMA,
        pltpu.SemaphoreType.DMA,
        pltpu.SemaphoreType.DMA,
        pltpu.SemaphoreType.DMA,
        pltpu.SemaphoreType.DMA,
        pltpu.SemaphoreType.DMA,
        pltpu.SemaphoreType.DMA,
        pltpu.SemaphoreType.DMA,
    ],
    compiler_params=pltpu.CompilerParams(use_tc_tiling_on_sc=False),
)
def _sc_lookup(tbl_hbm, v_hbm, ids_hbm, cnt_hbm, out_hbm,
               idx_v, cnt_v, rows_v, tbl_sh, vv,
               si0, si1, sc0, sc1, sg0, sg1, so0, so1):
    wid = lax.axis_index("s") * NC + lax.axis_index("c")
    row0 = wid * ROWS_W
    sem_i = (si0, si1)
    sem_c = (sc0, sc1)
    sem_g = (sg0, sg1)
    sem_o = (so0, so1)

    pltpu.sync_copy(v_hbm, vv)
    @pl.when(lax.axis_index("s") == 0)
    def _():
        pltpu.sync_copy(tbl_hbm, tbl_sh)

    plsc.subcore_barrier()
    vvecs = [vv[pl.ds(j * LANES, LANES)] for j in range(EMB // LANES)]

    def in_copies(k, b):
        r = row0 + k
        return [
            pltpu.make_async_copy(
                ids_hbm.at[r, pl.ds(0, IW)], idx_v.at[b, 0], sem_i[b]),
            pltpu.make_async_copy(
                ids_hbm.at[r, pl.ds(IW, IW2)],
                idx_v.at[b, 1, pl.ds(0, IW2)], sem_i[b]),
            pltpu.make_async_copy(
                cnt_hbm.at[r], cnt_v.at[b, pl.ds(0, L)], sem_c[b]),
        ]

    def gather_copies(b):
        return [
            pltpu.make_async_copy(
                tbl_sh.at[idx_v.at[b, 0]],
                rows_v.at[b, pl.ds(0, IW)], sem_g[b]),
            pltpu.make_async_copy(
                tbl_sh.at[idx_v.at[b, 1, pl.ds(0, IW2)]],
                rows_v.at[b, pl.ds(IW, IW2)], sem_g[b]),
        ]

    def out_copy(k, b):
        return pltpu.make_async_copy(
            rows_v.at[b], out_hbm.at[row0 + k], sem_o[b])

    def issue_in(k, b):
        for c in in_copies(k, b):
            c.start()

    def wait_in(k, b):
        for c in in_copies(k, b):
            c.wait()

    def issue_gather(b):
        for c in gather_copies(b):
            c.start()

    def wait_gather(b):
        for c in gather_copies(b):
            c.wait()

    def fma(b):
        def do_tokens(g, njs):
            cvec = cnt_v[b, pl.ds(g * LANES, LANES)]
            for j in range(njs):
                t = g * LANES + j
                cj = cvec[j]
                for q in range(EMB // LANES):
                    rows_v[b, t, pl.ds(q * LANES, LANES)] = (
                        rows_v[b, t, pl.ds(q * LANES, LANES)] + cj * vvecs[q])

        def grp_body(g, c):
            do_tokens(g, LANES)
            return c

        lax.fori_loop(0, GRPF, grp_body, 0)
        do_tokens(GRPF, TAIL)

    issue_in(0, 0)
    wait_in(0, 0)
    issue_gather(0)
    issue_in(1, 1)

    def body(m, carry):
        k0 = 2 * m
        k1 = k0 + 1
        k2 = k0 + 2
        k3 = k0 + 3

        wait_gather(0)
        wait_in(k1, 1)

        @pl.when(m > 0)
        def _():
            out_copy(k0 - 1, 1).wait()

        issue_gather(1)
        fma(0)
        out_copy(k0, 0).start()

        @pl.when(k2 < ROWS_W)
        def _():
            issue_in(k2, 0)

        wait_gather(1)

        @pl.when(k2 < ROWS_W)
        def _():
            wait_in(k2, 0)
            out_copy(k0, 0).wait()
            issue_gather(0)

        fma(1)
        out_copy(k1, 1).start()

        @pl.when(k3 < ROWS_W)
        def _():
            issue_in(k3, 1)

        return carry

    lax.fori_loop(0, ROWS_W // 2, body, 0)

    out_copy(ROWS_W - 2, 0).wait()
    out_copy(ROWS_W - 1, 1).wait()


def kernel(relic_ids, counters, emb_table, Wc, bc, Wf, bf):
    ids = relic_ids.astype(jnp.int32)
    cnt = counters.astype(jnp.float32)
    tbl, vrow = _prep(emb_table, Wc, bc.reshape(1, 8), Wf, bf.reshape(1, EMB))
    return _sc_lookup(tbl, vrow.reshape(EMB), ids, cnt)


# speedup vs baseline: 2.3800x; 2.3800x over previous
import functools

import jax
import jax.numpy as jnp
from jax import lax
from jax.experimental import pallas as pl
from jax.experimental.pallas import tpu as pltpu
from jax.experimental.pallas import tpu_sc as plsc

B, L = 4096, 200
EMB = 64
ID_DIM = EMB - 8
VOCAB = 201

NC, NS = 2, 16
NW = NC * NS
TOK = B * L
TOK_W = TOK // NW
IW = 128
IDXROWS = 4
CHUNK = IW * IDXROWS
NCHUNK = TOK_W // CHUNK
LANES = 16
GRP = CHUNK // LANES

assert TOK % NW == 0 and TOK_W % CHUNK == 0 and NCHUNK % 2 == 0


def _prep_body(emb_ref, wc_ref, bc_ref, wf_ref, bf_ref, tbl_ref, v_ref):
    wf = wf_ref[...]
    wf1 = wf[:, :ID_DIM]
    wf2 = wf[:, ID_DIM:]
    const = lax.dot_general(bc_ref[...], wf2, (((1,), (1,)), ((), ())),
                            preferred_element_type=jnp.float32) + bf_ref[...]
    tbl = lax.dot_general(emb_ref[...], wf1, (((1,), (1,)), ((), ())),
                          preferred_element_type=jnp.float32)
    tbl_ref[...] = tbl + const
    v_ref[...] = lax.dot_general(wc_ref[...], wf2, (((0,), (1,)), ((), ())),
                                 preferred_element_type=jnp.float32)


_prep = pl.pallas_call(
    _prep_body,
    out_shape=(
        jax.ShapeDtypeStruct((VOCAB, EMB), jnp.float32),
        jax.ShapeDtypeStruct((1, EMB), jnp.float32),
    ),
)


_sc_mesh = plsc.VectorSubcoreMesh(core_axis_name="c", subcore_axis_name="s")


@functools.partial(
    pl.kernel,
    out_type=jax.ShapeDtypeStruct((TOK, 2 * EMB), jnp.float32),
    mesh=_sc_mesh,
    scratch_types=[
        pltpu.VMEM((2, IDXROWS, IW), jnp.int32),
        pltpu.VMEM((2, CHUNK), jnp.float32),
        pltpu.VMEM((2, CHUNK, EMB), jnp.float32),
        pltpu.VMEM_SHARED((VOCAB, EMB), jnp.float32),
        pltpu.VMEM((EMB,), jnp.float32),
        pltpu.SemaphoreType.DMA,
        pltpu.SemaphoreType.DMA,
        pltpu.SemaphoreType.DMA,
        pltpu.SemaphoreType.DMA,
        pltpu.SemaphoreType.DMA,
        pltpu.SemaphoreType.DMA,
        pltpu.SemaphoreType.DMA,
        pltpu.SemaphoreType.DMA,
    ],
    compiler_params=pltpu.CompilerParams(use_tc_tiling_on_sc=False),
)
def _sc_lookup(tbl_hbm, v_hbm, ids2_hbm, cnt_hbm, out_hbm,
               idx_v, cnt_v, rows_v, tbl_v, vv,
               si0, si1, sc0, sc1, sg0, sg1, so0, so1):
    wid = lax.axis_index("s") * NC + lax.axis_index("c")
    base0 = wid * TOK_W
    row0 = wid * (TOK_W // IW)
    sem_i = (si0, si1)
    sem_c = (sc0, sc1)
    sem_g = (sg0, sg1)
    sem_o = (so0, so1)

    pltpu.sync_copy(v_hbm, vv)
    @pl.when(lax.axis_index("s") == 0)
    def _():
        pltpu.sync_copy(tbl_hbm, tbl_v)

    plsc.subcore_barrier()
    vvecs = [vv[pl.ds(j * LANES, LANES)] for j in range(EMB // LANES)]

    def in_copies(k, b):
        return (
            pltpu.make_async_copy(
                ids2_hbm.at[pl.ds(row0 + k * IDXROWS, IDXROWS)],
                idx_v.at[b], sem_i[b]),
            pltpu.make_async_copy(
                cnt_hbm.at[pl.ds(base0 + k * CHUNK, CHUNK)],
                cnt_v.at[b], sem_c[b]),
        )

    def gather_copies(b):
        return [
            pltpu.make_async_copy(
                tbl_v.at[idx_v.at[b, i]],
                rows_v.at[b, pl.ds(i * IW, IW)], sem_g[b])
            for i in range(IDXROWS)
        ]

    def out_copy(k, b):
        return pltpu.make_async_copy(
            rows_v.at[b],
            out_hbm.at[pl.ds(base0 + k * CHUNK, CHUNK), pl.ds(0, EMB)],
            sem_o[b])

    def issue_in(k, b):
        for c in in_copies(k, b):
            c.start()

    def wait_in(k, b):
        for c in in_copies(k, b):
            c.wait()

    def issue_gather(b):
        for c in gather_copies(b):
            c.start()

    def wait_gather(b):
        for c in gather_copies(b):
            c.wait()

    def fma(b):
        def grp_body(g, c):
            cvec = cnt_v[b, pl.ds(g * LANES, LANES)]
            for j in range(LANES):
                t = g * LANES + j
                cj = cvec[j]
                for q in range(EMB // LANES):
                    rows_v[b, t, pl.ds(q * LANES, LANES)] = (
                        rows_v[b, t, pl.ds(q * LANES, LANES)] + cj * vvecs[q])
            return c

        lax.fori_loop(0, GRP, grp_body, 0)

    issue_in(0, 0)
    wait_in(0, 0)
    issue_gather(0)
    issue_in(1, 1)

    def body(m, carry):
        k0 = 2 * m
        k1 = k0 + 1
        k2 = k0 + 2
        k3 = k0 + 3

        wait_gather(0)
        wait_in(k1, 1)

        @pl.when(m > 0)
        def _():
            out_copy(k0 - 1, 1).wait()

        issue_gather(1)
        fma(0)
        out_copy(k0, 0).start()

        @pl.when(k2 < NCHUNK)
        def _():
            issue_in(k2, 0)

        wait_gather(1)

        @pl.when(k2 < NCHUNK)
        def _():
            wait_in(k2, 0)
            out_copy(k0, 0).wait()
            issue_gather(0)

        fma(1)
        out_copy(k1, 1).start()

        @pl.when(k3 < NCHUNK)
        def _():
            issue_in(k3, 1)

        return carry

    lax.fori_loop(0, NCHUNK // 2, body, 0)

    out_copy(NCHUNK - 2, 0).wait()
    out_copy(NCHUNK - 1, 1).wait()


def kernel(relic_ids, counters, emb_table, Wc, bc, Wf, bf):
    ids2 = relic_ids.reshape(TOK // IW, IW).astype(jnp.int32)
    cnt = counters.reshape(TOK).astype(jnp.float32)
    tbl, vrow = _prep(emb_table, Wc, bc.reshape(1, 8), Wf, bf.reshape(1, EMB))
    out = _sc_lookup(tbl, vrow.reshape(EMB), ids2, cnt)
    return out.reshape(B, L, 2 * EMB)[:, :, :EMB]


# speedup vs baseline: 2.3990x; 1.0080x over previous
import functools

import jax
import jax.numpy as jnp
from jax import lax
from jax.experimental import pallas as pl
from jax.experimental.pallas import tpu as pltpu
from jax.experimental.pallas import tpu_sc as plsc

B, L = 4096, 200
EMB = 64
ID_DIM = EMB - 8
VOCAB = 201

NC, NS = 2, 16
NW = NC * NS
TOK = B * L
TOK_W = TOK // NW
IW = 128
IDXROWS = 4
CHUNK = IW * IDXROWS
NCHUNK = TOK_W // CHUNK
LANES = 16
GRP = CHUNK // LANES

assert TOK % NW == 0 and TOK_W % CHUNK == 0 and NCHUNK % 2 == 0


def _prep_body(emb_ref, wc_ref, bc_ref, wf_ref, bf_ref, tbl_ref, v_ref):
    wf = wf_ref[...]
    wf1 = wf[:, :ID_DIM]
    wf2 = wf[:, ID_DIM:]
    const = lax.dot_general(bc_ref[...], wf2, (((1,), (1,)), ((), ())),
                            preferred_element_type=jnp.float32) + bf_ref[...]
    tbl = lax.dot_general(emb_ref[...], wf1, (((1,), (1,)), ((), ())),
                          preferred_element_type=jnp.float32)
    tbl_ref[...] = tbl + const
    v_ref[...] = lax.dot_general(wc_ref[...], wf2, (((0,), (1,)), ((), ())),
                                 preferred_element_type=jnp.float32)


_prep = pl.pallas_call(
    _prep_body,
    out_shape=(
        jax.ShapeDtypeStruct((VOCAB, EMB), jnp.float32),
        jax.ShapeDtypeStruct((1, EMB), jnp.float32),
    ),
)


_sc_mesh = plsc.VectorSubcoreMesh(core_axis_name="c", subcore_axis_name="s")


@functools.partial(
    pl.kernel,
    out_type=jax.ShapeDtypeStruct((TOK, 2 * EMB), jnp.float32),
    mesh=_sc_mesh,
    scratch_types=[
        pltpu.VMEM((2, IDXROWS, IW), jnp.int32),
        pltpu.VMEM((2, CHUNK), jnp.float32),
        pltpu.VMEM((2, CHUNK, EMB), jnp.float32),
        pltpu.VMEM_SHARED((VOCAB, EMB), jnp.float32),
        pltpu.VMEM((EMB,), jnp.float32),
        pltpu.SemaphoreType.DMA,
        pltpu.SemaphoreType.DMA,
        pltpu.SemaphoreType.DMA,
        pltpu.SemaphoreType.DMA,
        pltpu.SemaphoreType.DMA,
        pltpu.SemaphoreType.DMA,
        pltpu.SemaphoreType.DMA,
        pltpu.SemaphoreType.DMA,
    ],
    compiler_params=pltpu.CompilerParams(use_tc_tiling_on_sc=False),
)
def _sc_lookup(tbl_hbm, v_hbm, ids2_hbm, cnt_hbm, out_hbm,
               idx_v, cnt_v, rows_v, tbl_v, vv,
               si0, si1, sc0, sc1, sg0, sg1, so0, so1):
    wid = lax.axis_index("s") * NC + lax.axis_index("c")
    base0 = wid * TOK_W
    row0 = wid * (TOK_W // IW)
    sem_i = (si0, si1)
    sem_c = (sc0, sc1)
    sem_g = (sg0, sg1)
    sem_o = (so0, so1)

    pltpu.sync_copy(v_hbm, vv)
    @pl.when(lax.axis_index("s") == 0)
    def _():
        pltpu.sync_copy(tbl_hbm, tbl_v)

    plsc.subcore_barrier()
    vvecs = [vv[pl.ds(j * LANES, LANES)] for j in range(EMB // LANES)]

    def in_copies(k, b):
        return (
            pltpu.make_async_copy(
                ids2_hbm.at[pl.ds(row0 + k * IDXROWS, IDXROWS)],
                idx_v.at[b], sem_i[b]),
            pltpu.make_async_copy(
                cnt_hbm.at[pl.ds(base0 + k * CHUNK, CHUNK)],
                cnt_v.at[b], sem_c[b]),
        )

    def gather_copies(b):
        return [
            pltpu.make_async_copy(
                tbl_v.at[idx_v.at[b, i]],
                rows_v.at[b, pl.ds(i * IW, IW)], sem_g[b])
            for i in range(IDXROWS)
        ]

    def out_copy(k, b):
        return pltpu.make_async_copy(
            rows_v.at[b],
            out_hbm.at[pl.ds(base0 + k * CHUNK, CHUNK), pl.ds(0, EMB)],
            sem_o[b])

    def issue_in(k, b):
        for c in in_copies(k, b):
            c.start()

    def wait_in(k, b):
        for c in in_copies(k, b):
            c.wait()

    def issue_gather(b):
        for c in gather_copies(b):
            c.start()

    def wait_gather(b):
        for c in gather_copies(b):
            c.wait()

    def fma(b):
        def grp_body(g, c):
            cvec = cnt_v[b, pl.ds(g * LANES, LANES)]
            for j in range(LANES):
                t = g * LANES + j
                cj = cvec[j]
                for q in range(EMB // LANES):
                    plsc.addupdate(
                        rows_v.at[b, t, pl.ds(q * LANES, LANES)],
                        cj * vvecs[q])
            return c

        lax.fori_loop(0, GRP, grp_body, 0)

    issue_in(0, 0)
    wait_in(0, 0)
    issue_gather(0)
    issue_in(1, 1)

    def body(m, carry):
        k0 = 2 * m
        k1 = k0 + 1
        k2 = k0 + 2
        k3 = k0 + 3

        wait_gather(0)
        wait_in(k1, 1)

        @pl.when(m > 0)
        def _():
            out_copy(k0 - 1, 1).wait()

        issue_gather(1)
        fma(0)
        out_copy(k0, 0).start()

        @pl.when(k2 < NCHUNK)
        def _():
            issue_in(k2, 0)

        wait_gather(1)

        @pl.when(k2 < NCHUNK)
        def _():
            wait_in(k2, 0)
            out_copy(k0, 0).wait()
            issue_gather(0)

        fma(1)
        out_copy(k1, 1).start()

        @pl.when(k3 < NCHUNK)
        def _():
            issue_in(k3, 1)

        return carry

    lax.fori_loop(0, NCHUNK // 2, body, 0)

    out_copy(NCHUNK - 2, 0).wait()
    out_copy(NCHUNK - 1, 1).wait()


def kernel(relic_ids, counters, emb_table, Wc, bc, Wf, bf):
    ids2 = relic_ids.reshape(TOK // IW, IW).astype(jnp.int32)
    cnt = counters.reshape(TOK).astype(jnp.float32)
    tbl, vrow = _prep(emb_table, Wc, bc.reshape(1, 8), Wf, bf.reshape(1, EMB))
    out = _sc_lookup(tbl, vrow.reshape(EMB), ids2, cnt)
    return out.reshape(B, L, 2 * EMB)[:, :, :EMB]
